# baseline (speedup 1.0000x reference)
import functools

import jax
import jax.numpy as jnp
from jax import lax
from jax.experimental import pallas as pl
from jax.experimental.pallas import tpu as pltpu
from jax.experimental.pallas import tpu_sc as plsc

N = 10000
E = 320000
D = 128

NC = 2
NS = 16

K = 8
KH = 4
C = K * 128
ROWS_PER_TILE = 160
NCHUNK = ROWS_PER_TILE // K
E_PAD = NS * ROWS_PER_TILE * 128

NPH = 5120
ZROWS = 80
ACC_PER_TILE = NPH // NS

_BCAST_DNUMS = lax.GatherDimensionNumbers(
    offset_dims=(), collapsed_slice_dims=(0,), start_index_map=(0,))


def _make_sc_body(rows_per_tile, core_row_stride, nchunk):
  def _gcn_sc_body(src_hbm, dst_hbm, w_hbm, table_hbm, out_hbm,
                   srcv, dstv, wv, rows_v, zbuf, acc, sem, isem):
    cid = lax.axis_index("c")
    sid = lax.axis_index("s")
    row_lo = cid * NPH

    zero16 = jnp.zeros((16,), jnp.float32)

    def zb(i, _):
        for f in range(D // 16):
            zbuf[i, pl.ds(f * 16, 16)] = zero16
        return 0

    lax.fori_loop(0, ZROWS, zb, 0)
    for z in range(ACC_PER_TILE // ZROWS):
        pltpu.sync_copy(zbuf, acc.at[pl.ds(sid * ACC_PER_TILE + z * ZROWS, ZROWS)])
    plsc.subcore_barrier()

    def chunk(k, _):
        row_base = cid * core_row_stride + sid * rows_per_tile + k * K
        pltpu.sync_copy(src_hbm.at[pl.ds(row_base, K)], srcv)
        pltpu.sync_copy(dst_hbm.at[pl.ds(row_base, K)], dstv)
        pltpu.sync_copy(w_hbm.at[pl.ds(row_base * 128, C)], wv)

        descs = [
            pltpu.async_copy(table_hbm.at[srcv.at[j]], rows_v.at[j % KH], sem)
            for j in range(KH)
        ]
        for j in range(K):
            jb = j % KH
            descs[j].wait()

            wbase = j * 128

            def grp(g, _):
                dst16 = dstv[j, pl.ds(g * 16, 16)]
                local = dst16 - row_lo
                inr = (local >= 0) & (local < NPH)
                dstv[j, pl.ds(g * 16, 16)] = jnp.where(inr, local, 0)
                wvec = jnp.where(inr, wv[pl.ds(wbase + g * 16, 16)], 0.0)
                for l in range(16):
                    ws = lax.gather(
                        wvec, jnp.full((16, 1), l, jnp.int32),
                        _BCAST_DNUMS, (1,),
                        mode=lax.GatherScatterMode.PROMISE_IN_BOUNDS)
                    e = g * 16 + l
                    for f in range(D // 16):
                        rows_v[jb, e, pl.ds(f * 16, 16)] = (
                            rows_v[jb, e, pl.ds(f * 16, 16)] * ws)
                return 0

            lax.fori_loop(0, 8, grp, 0)

            pltpu.sync_copy(rows_v.at[jb], acc.at[dstv.at[j]], add=True)
            if j + KH < K:
                descs.append(
                    pltpu.async_copy(table_hbm.at[srcv.at[j + KH]],
                                     rows_v.at[(j + KH) % KH], sem))
        return 0

    lax.fori_loop(0, nchunk, chunk, 0)
    plsc.subcore_barrier()

    pltpu.sync_copy(acc.at[pl.ds(sid * ACC_PER_TILE, ACC_PER_TILE)],
                    out_hbm.at[cid, pl.ds(sid * ACC_PER_TILE, ACC_PER_TILE)])

  return _gcn_sc_body


EPC = 180224
RPC = EPC // 128

_SC_SCRATCH = [
    pltpu.VMEM((K, 128), jnp.int32),
    pltpu.VMEM((K, 128), jnp.int32),
    pltpu.VMEM((C,), jnp.float32),
    pltpu.VMEM((KH, 128, D), jnp.float32),
    pltpu.VMEM((ZROWS, D), jnp.float32),
    pltpu.VMEM_SHARED((NPH, D), jnp.float32),
    pltpu.SemaphoreType.DMA,
    pltpu.SemaphoreType.DMA,
]

_gcn_sc_full = functools.partial(
    pl.kernel,
    out_type=pltpu.MemorySpace.HBM((NC, NPH, D), jnp.float32),
    mesh=plsc.VectorSubcoreMesh(core_axis_name="c", subcore_axis_name="s"),
    scratch_types=_SC_SCRATCH,
)(_make_sc_body(ROWS_PER_TILE, 0, NCHUNK))

_gcn_sc_part = functools.partial(
    pl.kernel,
    out_type=pltpu.MemorySpace.HBM((NC, NPH, D), jnp.float32),
    mesh=plsc.VectorSubcoreMesh(core_axis_name="c", subcore_axis_name="s"),
    scratch_types=_SC_SCRATCH,
)(_make_sc_body(RPC // NS, RPC, RPC // NS // K))


def _mm_body(a_ref, w_ref, b_ref, o_ref):
    f = (jnp.dot(a_ref[...], w_ref[...], preferred_element_type=jnp.float32)
         + b_ref[...])
    o_ref[...] = jnp.maximum(f, 0.0)


BLK = 1024


def _mm(a, weight, bias2d):
    return pl.pallas_call(
        _mm_body,
        grid=(NC * NPH // BLK,),
        in_specs=[
            pl.BlockSpec((BLK, D), lambda i: (i, 0)),
            pl.BlockSpec((D, D), lambda i: (0, 0)),
            pl.BlockSpec((1, D), lambda i: (0, 0)),
        ],
        out_specs=pl.BlockSpec((BLK, D), lambda i: (i, 0)),
        out_shape=jax.ShapeDtypeStruct((NC * NPH, D), jnp.float32),
    )(a, weight, bias2d)


def kernel(node_embeds, edge_indices, edge_weights, weight, bias):
    src = edge_indices[1].astype(jnp.int32)
    dst = edge_indices[0].astype(jnp.int32)
    w = edge_weights.astype(jnp.float32)

    key = (dst >= NPH).astype(jnp.int32)
    below = jnp.cumsum(1 - key) - 1
    above = jnp.cumsum(key) - 1
    cnt0 = below[-1] + 1
    cnt1 = E - cnt0
    overflow = (cnt0 > EPC) | (cnt1 > EPC)
    pos = jnp.where(key == 0, below, EPC + above)
    psrc = jnp.zeros((2 * EPC,), jnp.int32).at[pos].set(src, mode="drop")
    pdst = jnp.zeros((2 * EPC,), jnp.int32).at[pos].set(dst, mode="drop")
    pw = jnp.zeros((2 * EPC,), jnp.float32).at[pos].set(w, mode="drop")

    pad = E_PAD - E
    fsrc = jnp.concatenate([src, jnp.zeros((pad,), jnp.int32)]).reshape(-1, 128)
    fdst = jnp.concatenate([dst, jnp.zeros((pad,), jnp.int32)]).reshape(-1, 128)
    fw = jnp.concatenate([w, jnp.zeros((pad,), jnp.float32)])

    halves = lax.cond(
        overflow,
        lambda: _gcn_sc_full(fsrc, fdst, fw, node_embeds),
        lambda: _gcn_sc_part(psrc.reshape(-1, 128), pdst.reshape(-1, 128),
                             pw, node_embeds),
    )
    a = halves.reshape(NC * NPH, D)
    return _mm(a, weight, bias.reshape(1, D))[:N]


# speedup vs baseline: 5.8289x; 5.8289x over previous
import functools

import jax
import jax.numpy as jnp
from jax import lax
from jax.experimental import pallas as pl
from jax.experimental.pallas import tpu as pltpu
from jax.experimental.pallas import tpu_sc as plsc

N = 10000
E = 320000
D = 128

NC = 2
NS = 16

K = 16
KH = 4
C = K * 128
ROWS_PER_TILE = 160
NCHUNK = ROWS_PER_TILE // K
E_PAD = NS * ROWS_PER_TILE * 128

NPH = 5120
ZROWS = 80
ACC_PER_TILE = NPH // NS

_BCAST_DNUMS = lax.GatherDimensionNumbers(
    offset_dims=(), collapsed_slice_dims=(0,), start_index_map=(0,))


def _gcn_sc_body(src_hbm, dst_hbm, w_hbm, table_hbm, out_hbm,
                 srcv, dstv, wv, rows_v, zbuf, acc, sem):
    cid = lax.axis_index("c")
    sid = lax.axis_index("s")
    row_lo = cid * NPH

    zero16 = jnp.zeros((16,), jnp.float32)

    def zb(i, _):
        for f in range(D // 16):
            zbuf[i, pl.ds(f * 16, 16)] = zero16
        return 0

    lax.fori_loop(0, ZROWS, zb, 0)
    for z in range(ACC_PER_TILE // ZROWS):
        pltpu.sync_copy(zbuf, acc.at[pl.ds(sid * ACC_PER_TILE + z * ZROWS, ZROWS)])
    plsc.subcore_barrier()

    def chunk(k, _):
        row_base = sid * ROWS_PER_TILE + k * K
        pltpu.sync_copy(src_hbm.at[pl.ds(row_base, K)], srcv)
        pltpu.sync_copy(dst_hbm.at[pl.ds(row_base, K)], dstv)
        pltpu.sync_copy(w_hbm.at[pl.ds(row_base * 128, C)], wv)

        descs = [
            pltpu.async_copy(table_hbm.at[srcv.at[j]], rows_v.at[j % KH], sem)
            for j in range(KH)
        ]
        for j in range(K):
            jb = j % KH
            descs[j].wait()

            wbase = j * 128

            def grp(g, _):
                dst16 = dstv[j, pl.ds(g * 16, 16)]
                local = dst16 - row_lo
                inr = (local >= 0) & (local < NPH)
                dstv[j, pl.ds(g * 16, 16)] = jnp.where(inr, local, 0)
                wvec = jnp.where(inr, wv[pl.ds(wbase + g * 16, 16)], 0.0)
                for l in range(16):
                    ws = lax.gather(
                        wvec, jnp.full((16, 1), l, jnp.int32),
                        _BCAST_DNUMS, (1,),
                        mode=lax.GatherScatterMode.PROMISE_IN_BOUNDS)
                    e = g * 16 + l
                    for f in range(D // 16):
                        rows_v[jb, e, pl.ds(f * 16, 16)] = (
                            rows_v[jb, e, pl.ds(f * 16, 16)] * ws)
                return 0

            lax.fori_loop(0, 8, grp, 0)

            pltpu.sync_copy(rows_v.at[jb], acc.at[dstv.at[j]], add=True)
            if j + KH < K:
                descs.append(
                    pltpu.async_copy(table_hbm.at[srcv.at[j + KH]],
                                     rows_v.at[(j + KH) % KH], sem))
        return 0

    lax.fori_loop(0, NCHUNK, chunk, 0)
    plsc.subcore_barrier()

    pltpu.sync_copy(acc.at[pl.ds(sid * ACC_PER_TILE, ACC_PER_TILE)],
                    out_hbm.at[cid, pl.ds(sid * ACC_PER_TILE, ACC_PER_TILE)])


_gcn_sc = functools.partial(
    pl.kernel,
    out_type=pltpu.MemorySpace.HBM((NC, NPH, D), jnp.float32),
    mesh=plsc.VectorSubcoreMesh(core_axis_name="c", subcore_axis_name="s"),
    scratch_types=[
        pltpu.VMEM((K, 128), jnp.int32),
        pltpu.VMEM((K, 128), jnp.int32),
        pltpu.VMEM((C,), jnp.float32),
        pltpu.VMEM((KH, 128, D), jnp.float32),
        pltpu.VMEM((ZROWS, D), jnp.float32),
        pltpu.VMEM_SHARED((NPH, D), jnp.float32),
        pltpu.SemaphoreType.DMA,
    ],
)(_gcn_sc_body)


def _mm_body(a_ref, w_ref, b_ref, o_ref):
    f = (jnp.dot(a_ref[...], w_ref[...], preferred_element_type=jnp.float32)
         + b_ref[...])
    o_ref[...] = jnp.maximum(f, 0.0)


BLK = 1024


def _mm(a, weight, bias2d):
    return pl.pallas_call(
        _mm_body,
        grid=(NC * NPH // BLK,),
        in_specs=[
            pl.BlockSpec((BLK, D), lambda i: (i, 0)),
            pl.BlockSpec((D, D), lambda i: (0, 0)),
            pl.BlockSpec((1, D), lambda i: (0, 0)),
        ],
        out_specs=pl.BlockSpec((BLK, D), lambda i: (i, 0)),
        out_shape=jax.ShapeDtypeStruct((NC * NPH, D), jnp.float32),
    )(a, weight, bias2d)


def kernel(node_embeds, edge_indices, edge_weights, weight, bias):
    src = edge_indices[1].astype(jnp.int32)
    dst = edge_indices[0].astype(jnp.int32)
    w = edge_weights.astype(jnp.float32)
    pad = E_PAD - E
    src = jnp.concatenate([src, jnp.zeros((pad,), jnp.int32)]).reshape(-1, 128)
    dst = jnp.concatenate([dst, jnp.zeros((pad,), jnp.int32)]).reshape(-1, 128)
    w = jnp.concatenate([w, jnp.zeros((pad,), jnp.float32)])
    halves = _gcn_sc(src, dst, w, node_embeds)
    a = halves.reshape(NC * NPH, D)
    return _mm(a, weight, bias.reshape(1, D))[:N]


# speedup vs baseline: 5.8810x; 1.0089x over previous
import functools

import jax
import jax.numpy as jnp
from jax import lax
from jax.experimental import pallas as pl
from jax.experimental.pallas import tpu as pltpu
from jax.experimental.pallas import tpu_sc as plsc

N = 10000
E = 320000
D = 128

NC = 2
NS = 16

K = 16
KH = 4
C = K * 128
ROWS_PER_TILE = 160
NCHUNK = ROWS_PER_TILE // K
E_PAD = NS * ROWS_PER_TILE * 128

NPH = 5120
ZROWS = 80
ACC_PER_TILE = NPH // NS

_BCAST_DNUMS = lax.GatherDimensionNumbers(
    offset_dims=(), collapsed_slice_dims=(0,), start_index_map=(0,))


def _gcn_sc_body(src_hbm, dst_hbm, w_hbm, table_hbm, out_hbm,
                 srcv, dstv, wv, rows_v, zbuf, acc, sem):
    cid = lax.axis_index("c")
    sid = lax.axis_index("s")
    row_lo = cid * NPH

    zero16 = jnp.zeros((16,), jnp.float32)

    def zb(i, _):
        for f in range(D // 16):
            zbuf[i, pl.ds(f * 16, 16)] = zero16
        return 0

    lax.fori_loop(0, ZROWS, zb, 0)
    for z in range(ACC_PER_TILE // ZROWS):
        pltpu.sync_copy(zbuf, acc.at[pl.ds(sid * ACC_PER_TILE + z * ZROWS, ZROWS)])
    plsc.subcore_barrier()

    def chunk(k, _):
        row_base = sid * ROWS_PER_TILE + k * K
        pltpu.sync_copy(src_hbm.at[pl.ds(row_base, K)], srcv)

        descs = [
            pltpu.async_copy(table_hbm.at[srcv.at[j]], rows_v.at[j % KH], sem)
            for j in range(KH)
        ]
        pltpu.sync_copy(dst_hbm.at[pl.ds(row_base, K)], dstv)
        pltpu.sync_copy(w_hbm.at[pl.ds(row_base * 128, C)], wv)
        for j in range(K):
            jb = j % KH
            descs[j].wait()

            wbase = j * 128

            def grp(g, _):
                dst16 = dstv[j, pl.ds(g * 16, 16)]
                local = dst16 - row_lo
                inr = (local >= 0) & (local < NPH)
                dstv[j, pl.ds(g * 16, 16)] = jnp.where(inr, local, 0)
                wvec = jnp.where(inr, wv[pl.ds(wbase + g * 16, 16)], 0.0)
                for l in range(16):
                    ws = lax.gather(
                        wvec, jnp.full((16, 1), l, jnp.int32),
                        _BCAST_DNUMS, (1,),
                        mode=lax.GatherScatterMode.PROMISE_IN_BOUNDS)
                    e = g * 16 + l
                    for f in range(D // 16):
                        rows_v[jb, e, pl.ds(f * 16, 16)] = (
                            rows_v[jb, e, pl.ds(f * 16, 16)] * ws)
                return 0

            lax.fori_loop(0, 8, grp, 0)

            pltpu.sync_copy(rows_v.at[jb], acc.at[dstv.at[j]], add=True)
            if j + KH < K:
                descs.append(
                    pltpu.async_copy(table_hbm.at[srcv.at[j + KH]],
                                     rows_v.at[(j + KH) % KH], sem))
        return 0

    lax.fori_loop(0, NCHUNK, chunk, 0)
    plsc.subcore_barrier()

    pltpu.sync_copy(acc.at[pl.ds(sid * ACC_PER_TILE, ACC_PER_TILE)],
                    out_hbm.at[cid, pl.ds(sid * ACC_PER_TILE, ACC_PER_TILE)])


_gcn_sc = functools.partial(
    pl.kernel,
    out_type=pltpu.MemorySpace.HBM((NC, NPH, D), jnp.float32),
    mesh=plsc.VectorSubcoreMesh(core_axis_name="c", subcore_axis_name="s"),
    scratch_types=[
        pltpu.VMEM((K, 128), jnp.int32),
        pltpu.VMEM((K, 128), jnp.int32),
        pltpu.VMEM((C,), jnp.float32),
        pltpu.VMEM((KH, 128, D), jnp.float32),
        pltpu.VMEM((ZROWS, D), jnp.float32),
        pltpu.VMEM_SHARED((NPH, D), jnp.float32),
        pltpu.SemaphoreType.DMA,
    ],
)(_gcn_sc_body)


def _mm_body(a_ref, w_ref, b_ref, o_ref):
    f = (jnp.dot(a_ref[...], w_ref[...], preferred_element_type=jnp.float32)
         + b_ref[...])
    o_ref[...] = jnp.maximum(f, 0.0)


BLK = 1024


def _mm(a, weight, bias2d):
    return pl.pallas_call(
        _mm_body,
        grid=(NC * NPH // BLK,),
        in_specs=[
            pl.BlockSpec((BLK, D), lambda i: (i, 0)),
            pl.BlockSpec((D, D), lambda i: (0, 0)),
            pl.BlockSpec((1, D), lambda i: (0, 0)),
        ],
        out_specs=pl.BlockSpec((BLK, D), lambda i: (i, 0)),
        out_shape=jax.ShapeDtypeStruct((NC * NPH, D), jnp.float32),
    )(a, weight, bias2d)


def kernel(node_embeds, edge_indices, edge_weights, weight, bias):
    src = edge_indices[1].astype(jnp.int32)
    dst = edge_indices[0].astype(jnp.int32)
    w = edge_weights.astype(jnp.float32)
    pad = E_PAD - E
    src = jnp.concatenate([src, jnp.zeros((pad,), jnp.int32)]).reshape(-1, 128)
    dst = jnp.concatenate([dst, jnp.zeros((pad,), jnp.int32)]).reshape(-1, 128)
    w = jnp.concatenate([w, jnp.zeros((pad,), jnp.float32)])
    halves = _gcn_sc(src, dst, w, node_embeds)
    a = halves.reshape(NC * NPH, D)
    return _mm(a, weight, bias.reshape(1, D))[:N]
